# baseline (speedup 1.0000x reference)
import functools

import jax
import jax.numpy as jnp
from jax import lax
from jax.experimental import pallas as pl
from jax.experimental.pallas import tpu as pltpu
from jax.experimental.pallas import tpu_sc as plsc

_TM = 512
_TKC = 2048


_WINDOWS = ((0, 2736), (2736, 5472), (5472, 8192))


_TS = 64


def _vq_tc_body(zsq_ref, z_ref, cb_ref, csq_ref, idx_ref, loss_ref, d_ref):
    i = pl.program_id(0)
    kk = cb_ref.shape[0]
    tm = z_ref.shape[0]
    tkc = min(_TKC, kk)
    zneg = z_ref[...] * (-2.0)
    zsq = zsq_ref[...]
    for c in range(kk // tkc):
        cb = cb_ref[c * tkc:(c + 1) * tkc, :]
        zc = lax.dot_general(zneg, cb, (((1,), (1,)), ((), ())),
                             preferred_element_type=jnp.float32)
        d_ref[:, c * tkc:(c + 1) * tkc] = \
            (zsq + zc) + csq_ref[:, c * tkc:(c + 1) * tkc]

    inf = jnp.float32(jnp.inf)
    ts = _TS
    lane_i = lax.broadcasted_iota(jnp.int32, (ts, 128), 1)
    lane_f = lane_i.astype(jnp.float32)
    win_groups = []
    for (lo, hi) in _WINDOWS:
        groups = []
        for g in range(lo // 128, -(-hi // 128)):
            a, b = g * 128, (g + 1) * 128
            if a >= lo and b <= hi:
                groups.append((g, None))
            else:
                groups.append((g, (lane_i >= (lo - a)) if a < lo
                               else (lane_i < (hi - a))))
        win_groups.append(groups)

    bsum = jnp.float32(0.0)
    for t in range(tm // ts):
        r0 = t * ts
        acc_cmp = jnp.full((ts, 1), inf, jnp.float32)
        acc_val = jnp.full((ts, 1), inf, jnp.float32)
        acc_idx = jnp.zeros((ts, 1), jnp.float32)
        for groups in win_groups:
            runv = jnp.full((ts, 128), inf, jnp.float32)
            rung = jnp.zeros((ts, 128), jnp.float32)
            for (g, mask) in groups:
                dsub = d_ref[r0:r0 + ts, g * 128:(g + 1) * 128]
                if mask is not None:
                    dsub = jnp.where(mask, dsub, inf)
                pred = dsub < runv
                runv = jnp.where(pred, dsub, runv)
                rung = jnp.where(pred, jnp.float32(g), rung)
            cmin = jnp.min(runv, axis=1, keepdims=True)
            colf = rung * jnp.float32(128.0) + lane_f
            cidx = jnp.min(jnp.where(runv == cmin, colf, jnp.float32(kk)),
                           axis=1, keepdims=True)
            take = cmin < acc_cmp
            acc_cmp = jnp.where(
                take, cmin.astype(jnp.bfloat16).astype(jnp.float32), acc_cmp)
            acc_val = jnp.where(take, cmin, acc_val)
            acc_idx = jnp.where(take, cidx, acc_idx)
        idx_ref[r0:r0 + ts, :] = acc_idx.astype(jnp.int32)
        bsum += jnp.sum(acc_val)

    @pl.when(i == 0)
    def _():
        loss_ref[0, 0] = 0.0

    loss_ref[0, 0] += bsum


def _tc_argmin(flat, zsq, cb, csq):
    m, d = flat.shape
    kk = cb.shape[0]
    tm = _TM if m % _TM == 0 else m
    return pl.pallas_call(
        _vq_tc_body,
        grid=(m // tm,),
        in_specs=[
            pl.BlockSpec((tm, 1), lambda i: (i, 0)),
            pl.BlockSpec((tm, d), lambda i: (i, 0)),
            pl.BlockSpec((kk, d), lambda i: (0, 0)),
            pl.BlockSpec((1, kk), lambda i: (0, 0)),
        ],
        out_specs=[
            pl.BlockSpec((tm, 1), lambda i: (i, 0)),
            pl.BlockSpec((1, 1), lambda i: (0, 0), memory_space=pltpu.SMEM),
        ],
        out_shape=[
            jax.ShapeDtypeStruct((m, 1), jnp.int32),
            jax.ShapeDtypeStruct((1, 1), jnp.float32),
        ],
        scratch_shapes=[pltpu.VMEM((tm, kk), jnp.float32)],
    )(zsq, flat, cb, csq)


def _sc_gather(codebook, idx):
    kk, d = codebook.shape
    b = idx.shape[0]
    info = plsc.get_sparse_core_info()
    nw = info.num_cores * info.num_subcores
    bpw = b // nw
    ch = 128
    nch = bpw // ch
    mesh = plsc.VectorSubcoreMesh(core_axis_name="c", subcore_axis_name="s")

    @functools.partial(
        pl.kernel, mesh=mesh,
        out_type=jax.ShapeDtypeStruct((b, d), jnp.float32),
        scratch_types=[
            pltpu.VMEM((bpw,), jnp.int32),
            pltpu.VMEM((ch, d), jnp.float32),
            pltpu.VMEM((ch, d), jnp.float32),
            pltpu.SemaphoreType.DMA,
            pltpu.SemaphoreType.DMA,
        ],
    )
    def gather_k(cb_hbm, idx_hbm, out_hbm, idx_v, buf0, buf1, sem0, sem1):
        wid = lax.axis_index("s") * info.num_cores + lax.axis_index("c")
        base = wid * bpw
        pltpu.sync_copy(idx_hbm.at[pl.ds(base, bpw)], idx_v)
        bufs, sems = (buf0, buf1), (sem0, sem1)
        cps = [None] * nch
        cps[0] = pltpu.async_copy(
            cb_hbm.at[idx_v.at[pl.ds(0, ch)]], bufs[0], sems[0])
        for c in range(nch):
            if c + 1 < nch:
                cps[c + 1] = pltpu.async_copy(
                    cb_hbm.at[idx_v.at[pl.ds((c + 1) * ch, ch)]],
                    bufs[(c + 1) % 2], sems[(c + 1) % 2])
            cps[c].wait()
            pltpu.sync_copy(bufs[c % 2], out_hbm.at[pl.ds(base + c * ch, ch)])

    return gather_k(codebook, idx)


def kernel(z_e, codebook):
    bb, tt, d = z_e.shape
    flat = z_e.reshape(-1, d)
    zsq = jnp.sum(flat ** 2, axis=-1, keepdims=True)
    csq = jnp.sum(codebook ** 2, axis=-1, keepdims=True).T
    half = flat.shape[0] // 2
    idxs, losses, zqs = [], [], []
    for h in range(2):
        sl = slice(h * half, (h + 1) * half)
        idx2h, lossh = _tc_argmin(flat[sl], zsq[sl], codebook, csq)
        idxs.append(idx2h)
        losses.append(lossh)
        zqs.append(_sc_gather(codebook, idx2h.reshape(-1)))
    idx = jnp.concatenate(idxs, axis=0).reshape(-1)
    loss = losses[0] + losses[1]
    zq = jnp.concatenate(zqs, axis=0)
    m = loss[0, 0] / jnp.float32(flat.shape[0] * d)
    vq_loss = m + jnp.float32(0.25) * m
    return idx.reshape(bb, tt), zq.reshape(bb, tt, d), vq_loss


# speedup vs baseline: 1.2035x; 1.2035x over previous
import functools

import jax
import jax.numpy as jnp
from jax import lax
from jax.experimental import pallas as pl
from jax.experimental.pallas import tpu as pltpu
from jax.experimental.pallas import tpu_sc as plsc

_TM = 1024
_TKC = 2048


_WINDOWS = ((0, 2736), (2736, 5472), (5472, 8192))


_TS = 64


def _vq_tc_body(zsq_ref, z_ref, cb_ref, csq_ref, idx_ref, loss_ref, d_ref):
    i = pl.program_id(0)
    kk = cb_ref.shape[0]
    tm = z_ref.shape[0]
    tkc = min(_TKC, kk)
    zneg = z_ref[...] * (-2.0)
    zsq = zsq_ref[...]
    for c in range(kk // tkc):
        cb = cb_ref[c * tkc:(c + 1) * tkc, :]
        zc = lax.dot_general(zneg, cb, (((1,), (1,)), ((), ())),
                             preferred_element_type=jnp.float32)
        d_ref[:, c * tkc:(c + 1) * tkc] = \
            (zsq + zc) + csq_ref[:, c * tkc:(c + 1) * tkc]

    inf = jnp.float32(jnp.inf)
    ts = _TS
    lane_i = lax.broadcasted_iota(jnp.int32, (ts, 128), 1)
    lane_f = lane_i.astype(jnp.float32)
    win_groups = []
    for (lo, hi) in _WINDOWS:
        groups = []
        for g in range(lo // 128, -(-hi // 128)):
            a, b = g * 128, (g + 1) * 128
            if a >= lo and b <= hi:
                groups.append((g, None))
            else:
                groups.append((g, (lane_i >= (lo - a)) if a < lo
                               else (lane_i < (hi - a))))
        win_groups.append(groups)

    bsum = jnp.float32(0.0)
    for t in range(tm // ts):
        r0 = t * ts
        acc_cmp = jnp.full((ts, 1), inf, jnp.float32)
        acc_val = jnp.full((ts, 1), inf, jnp.float32)
        acc_idx = jnp.zeros((ts, 1), jnp.float32)
        for groups in win_groups:
            runv = jnp.full((ts, 128), inf, jnp.float32)
            rung = jnp.zeros((ts, 128), jnp.float32)
            for (g, mask) in groups:
                dsub = d_ref[r0:r0 + ts, g * 128:(g + 1) * 128]
                if mask is not None:
                    dsub = jnp.where(mask, dsub, inf)
                pred = dsub < runv
                runv = jnp.where(pred, dsub, runv)
                rung = jnp.where(pred, jnp.float32(g), rung)
            cmin = jnp.min(runv, axis=1, keepdims=True)
            colf = rung * jnp.float32(128.0) + lane_f
            cidx = jnp.min(jnp.where(runv == cmin, colf, jnp.float32(kk)),
                           axis=1, keepdims=True)
            take = cmin < acc_cmp
            acc_cmp = jnp.where(
                take, cmin.astype(jnp.bfloat16).astype(jnp.float32), acc_cmp)
            acc_val = jnp.where(take, cmin, acc_val)
            acc_idx = jnp.where(take, cidx, acc_idx)
        idx_ref[r0:r0 + ts, :] = acc_idx.astype(jnp.int32)
        bsum += jnp.sum(acc_val)

    @pl.when(i == 0)
    def _():
        loss_ref[0, 0] = 0.0

    loss_ref[0, 0] += bsum


def _tc_argmin(flat, zsq, cb, csq):
    m, d = flat.shape
    kk = cb.shape[0]
    tm = _TM if m % _TM == 0 else m
    return pl.pallas_call(
        _vq_tc_body,
        grid=(m // tm,),
        in_specs=[
            pl.BlockSpec((tm, 1), lambda i: (i, 0)),
            pl.BlockSpec((tm, d), lambda i: (i, 0)),
            pl.BlockSpec((kk, d), lambda i: (0, 0)),
            pl.BlockSpec((1, kk), lambda i: (0, 0)),
        ],
        out_specs=[
            pl.BlockSpec((tm, 1), lambda i: (i, 0)),
            pl.BlockSpec((1, 1), lambda i: (0, 0), memory_space=pltpu.SMEM),
        ],
        out_shape=[
            jax.ShapeDtypeStruct((m, 1), jnp.int32),
            jax.ShapeDtypeStruct((1, 1), jnp.float32),
        ],
        scratch_shapes=[pltpu.VMEM((tm, kk), jnp.float32)],
    )(zsq, flat, cb, csq)


def _sc_gather(codebook, idx):
    kk, d = codebook.shape
    b = idx.shape[0]
    info = plsc.get_sparse_core_info()
    nw = info.num_cores * info.num_subcores
    bpw = b // nw
    ch = 128
    nch = bpw // ch
    mesh = plsc.VectorSubcoreMesh(core_axis_name="c", subcore_axis_name="s")

    @functools.partial(
        pl.kernel, mesh=mesh,
        out_type=jax.ShapeDtypeStruct((b, d), jnp.float32),
        scratch_types=[
            pltpu.VMEM((bpw,), jnp.int32),
            pltpu.VMEM((ch, d), jnp.float32),
            pltpu.VMEM((ch, d), jnp.float32),
            pltpu.SemaphoreType.DMA,
            pltpu.SemaphoreType.DMA,
        ],
    )
    def gather_k(cb_hbm, idx_hbm, out_hbm, idx_v, buf0, buf1, sem0, sem1):
        wid = lax.axis_index("s") * info.num_cores + lax.axis_index("c")
        base = wid * bpw
        pltpu.sync_copy(idx_hbm.at[pl.ds(base, bpw)], idx_v)
        bufs, sems = (buf0, buf1), (sem0, sem1)
        cps = [None] * nch
        cps[0] = pltpu.async_copy(
            cb_hbm.at[idx_v.at[pl.ds(0, ch)]], bufs[0], sems[0])
        for c in range(nch):
            if c + 1 < nch:
                cps[c + 1] = pltpu.async_copy(
                    cb_hbm.at[idx_v.at[pl.ds((c + 1) * ch, ch)]],
                    bufs[(c + 1) % 2], sems[(c + 1) % 2])
            cps[c].wait()
            pltpu.sync_copy(bufs[c % 2], out_hbm.at[pl.ds(base + c * ch, ch)])

    return gather_k(codebook, idx)


def kernel(z_e, codebook):
    bb, tt, d = z_e.shape
    flat = z_e.reshape(-1, d)
    zsq = jnp.sum(flat ** 2, axis=-1, keepdims=True)
    csq = jnp.sum(codebook ** 2, axis=-1, keepdims=True).T
    idx2, loss = _tc_argmin(flat, zsq, codebook, csq)
    idx = idx2.reshape(-1)
    zq = _sc_gather(codebook, idx)
    m = loss[0, 0] / jnp.float32(flat.shape[0] * d)
    vq_loss = m + jnp.float32(0.25) * m
    return idx.reshape(bb, tt), zq.reshape(bb, tt, d), vq_loss


# speedup vs baseline: 1.2172x; 1.0114x over previous
import functools

import jax
import jax.numpy as jnp
from jax import lax
from jax.experimental import pallas as pl
from jax.experimental.pallas import tpu as pltpu
from jax.experimental.pallas import tpu_sc as plsc

_TM = 1024
_TKC = 2048


_WINDOWS = ((0, 2736), (2736, 5472), (5472, 8192))


_TS = 64


def _vq_tc_body(zsq_ref, z_ref, cb_ref, csq_ref, idx_ref, loss_ref, d_ref):
    i = pl.program_id(0)
    kk = cb_ref.shape[0]
    tm = z_ref.shape[0]
    tkc = min(_TKC, kk)
    zneg = z_ref[...] * (-2.0)
    zsq = zsq_ref[...]
    for c in range(kk // tkc):
        cb = cb_ref[c * tkc:(c + 1) * tkc, :]
        zc = lax.dot_general(zneg, cb, (((1,), (1,)), ((), ())),
                             preferred_element_type=jnp.float32)
        d_ref[:, c * tkc:(c + 1) * tkc] = \
            (zsq + zc) + csq_ref[:, c * tkc:(c + 1) * tkc]

    inf = jnp.float32(jnp.inf)
    ts = _TS
    lane_i = lax.broadcasted_iota(jnp.int32, (ts, 128), 1)
    lane_f = lane_i.astype(jnp.float32)
    win_groups = []
    for (lo, hi) in _WINDOWS:
        groups = []
        for g in range(lo // 128, -(-hi // 128)):
            a, b = g * 128, (g + 1) * 128
            if a >= lo and b <= hi:
                groups.append((g, None))
            else:
                groups.append((g, (lane_i >= (lo - a)) if a < lo
                               else (lane_i < (hi - a))))
        win_groups.append(groups)

    bsum = jnp.float32(0.0)
    rows = []
    for t in range(tm // ts):
        r0 = t * ts
        acc_cmp = jnp.full((ts, 1), inf, jnp.float32)
        acc_val = jnp.full((ts, 1), inf, jnp.float32)
        acc_idx = jnp.zeros((ts, 1), jnp.float32)
        for groups in win_groups:
            runv = jnp.full((ts, 128), inf, jnp.float32)
            rung = jnp.zeros((ts, 128), jnp.float32)
            for (g, mask) in groups:
                dsub = d_ref[r0:r0 + ts, g * 128:(g + 1) * 128]
                if mask is not None:
                    dsub = jnp.where(mask, dsub, inf)
                pred = dsub < runv
                runv = jnp.where(pred, dsub, runv)
                rung = jnp.where(pred, jnp.float32(g), rung)
            cmin = jnp.min(runv, axis=1, keepdims=True)
            colf = rung * jnp.float32(128.0) + lane_f
            cidx = jnp.min(jnp.where(runv == cmin, colf, jnp.float32(kk)),
                           axis=1, keepdims=True)
            take = cmin < acc_cmp
            acc_cmp = jnp.where(
                take, cmin.astype(jnp.bfloat16).astype(jnp.float32), acc_cmp)
            acc_val = jnp.where(take, cmin, acc_val)
            acc_idx = jnp.where(take, cidx, acc_idx)
        rows.append(jnp.transpose(acc_idx.astype(jnp.int32), (1, 0)))
        bsum += jnp.sum(acc_val)

    per_row = 128 // _TS
    paired = [jnp.concatenate(rows[k * per_row:(k + 1) * per_row], axis=1)
              for k in range(tm // 128)]
    idx_ref[...] = jnp.concatenate(paired, axis=0)

    @pl.when(i == 0)
    def _():
        loss_ref[0, 0] = 0.0

    loss_ref[0, 0] += bsum


def _tc_argmin(flat, zsq, cb, csq):
    m, d = flat.shape
    kk = cb.shape[0]
    tm = _TM if m % _TM == 0 else m
    return pl.pallas_call(
        _vq_tc_body,
        grid=(m // tm,),
        in_specs=[
            pl.BlockSpec((tm, 1), lambda i: (i, 0)),
            pl.BlockSpec((tm, d), lambda i: (i, 0)),
            pl.BlockSpec((kk, d), lambda i: (0, 0)),
            pl.BlockSpec((1, kk), lambda i: (0, 0)),
        ],
        out_specs=[
            pl.BlockSpec((tm // 128, 128), lambda i: (i, 0)),
            pl.BlockSpec((1, 1), lambda i: (0, 0), memory_space=pltpu.SMEM),
        ],
        out_shape=[
            jax.ShapeDtypeStruct((m // 128, 128), jnp.int32),
            jax.ShapeDtypeStruct((1, 1), jnp.float32),
        ],
        scratch_shapes=[pltpu.VMEM((tm, kk), jnp.float32)],
    )(zsq, flat, cb, csq)


def _sc_gather(codebook, idx):
    kk, d = codebook.shape
    b = idx.shape[0]
    info = plsc.get_sparse_core_info()
    nw = info.num_cores * info.num_subcores
    bpw = b // nw
    ch = 128
    nch = bpw // ch
    mesh = plsc.VectorSubcoreMesh(core_axis_name="c", subcore_axis_name="s")

    @functools.partial(
        pl.kernel, mesh=mesh,
        out_type=jax.ShapeDtypeStruct((b, d), jnp.float32),
        scratch_types=[
            pltpu.VMEM((bpw,), jnp.int32),
            pltpu.VMEM((ch, d), jnp.float32),
            pltpu.VMEM((ch, d), jnp.float32),
            pltpu.SemaphoreType.DMA,
            pltpu.SemaphoreType.DMA,
        ],
    )
    def gather_k(cb_hbm, idx_hbm, out_hbm, idx_v, buf0, buf1, sem0, sem1):
        wid = lax.axis_index("s") * info.num_cores + lax.axis_index("c")
        base = wid * bpw
        pltpu.sync_copy(idx_hbm.at[pl.ds(base, bpw)], idx_v)
        bufs, sems = (buf0, buf1), (sem0, sem1)
        cps = [None] * nch
        cps[0] = pltpu.async_copy(
            cb_hbm.at[idx_v.at[pl.ds(0, ch)]], bufs[0], sems[0])
        for c in range(nch):
            if c + 1 < nch:
                cps[c + 1] = pltpu.async_copy(
                    cb_hbm.at[idx_v.at[pl.ds((c + 1) * ch, ch)]],
                    bufs[(c + 1) % 2], sems[(c + 1) % 2])
            cps[c].wait()
            pltpu.sync_copy(bufs[c % 2], out_hbm.at[pl.ds(base + c * ch, ch)])

    return gather_k(codebook, idx)


def kernel(z_e, codebook):
    bb, tt, d = z_e.shape
    flat = z_e.reshape(-1, d)
    zsq = jnp.sum(flat ** 2, axis=-1, keepdims=True)
    csq = jnp.sum(codebook ** 2, axis=-1, keepdims=True).T
    idx2, loss = _tc_argmin(flat, zsq, codebook, csq)
    idx = idx2.reshape(-1)
    zq = _sc_gather(codebook, idx)
    m = loss[0, 0] / jnp.float32(flat.shape[0] * d)
    vq_loss = m + jnp.float32(0.25) * m
    return idx.reshape(bb, tt), zq.reshape(bb, tt, d), vq_loss
